# baseline (speedup 1.0000x reference)
import functools

import jax
import jax.numpy as jnp
from jax import lax
from jax.experimental import pallas as pl
from jax.experimental.pallas import tpu as pltpu
from jax.experimental.pallas import tpu_sc as plsc

_N_E = 8192
_E_DIM = 256
_BETA = 0.25

_BM = 2048
_BN = 2048
_NJ = _N_E // _BN


_NI = _N_E // _BM
_NIO = 1
_NII = _NI // _NIO


def _dist_argmin_kernel(l_ref, e_ref, t_ref, col_ref, out_ref,
                        s_ref, m_ref, i_ref):
    j = pl.program_id(1)
    row = pl.program_id(2) * _BM

    @pl.when(j == 0)
    def _():
        lb = l_ref[...]
        s_ref[pl.ds(row, _BM), :] = jnp.sum(lb * lb, axis=1, keepdims=True)

    lb = l_ref[...]
    mm2 = lax.dot_general(lb + lb, e_ref[...],
                          (((1,), (1,)), ((), ())),
                          preferred_element_type=jnp.float32)
    d = (s_ref[pl.ds(row, _BM), :] + t_ref[...]) - mm2
    bmin = jnp.min(d, axis=1, keepdims=True)
    bidxf = jnp.min(jnp.where(d == bmin, col_ref[...], jnp.float32(1e9)),
                    axis=1, keepdims=True)

    @pl.when(j == 0)
    def _():
        m_ref[pl.ds(row, _BM), :] = bmin
        i_ref[pl.ds(row, _BM), :] = bidxf

    @pl.when(j > 0)
    def _():
        upd = bmin < m_ref[pl.ds(row, _BM), :]
        m_ref[pl.ds(row, _BM), :] = jnp.where(upd, bmin, m_ref[pl.ds(row, _BM), :])
        i_ref[pl.ds(row, _BM), :] = jnp.where(upd, bidxf, i_ref[pl.ds(row, _BM), :])

    @pl.when(j == _NJ - 1)
    def _():
        out_ref[...] = i_ref[pl.ds(row, _BM), :].astype(jnp.int32)


def _argmin_indices(latent, emb, t, colf):
    return pl.pallas_call(
        _dist_argmin_kernel,
        grid=(_NIO, _NJ, _NII),
        in_specs=[
            pl.BlockSpec((_BM, _E_DIM), lambda io, j, ii: (io * _NII + ii, 0)),
            pl.BlockSpec((_BN, _E_DIM), lambda io, j, ii: (j, 0)),
            pl.BlockSpec((1, _BN), lambda io, j, ii: (0, j)),
            pl.BlockSpec((1, _BN), lambda io, j, ii: (0, j)),
        ],
        out_specs=pl.BlockSpec((_BM, 1), lambda io, j, ii: (io * _NII + ii, 0)),
        out_shape=jax.ShapeDtypeStruct((_N_E, 1), jnp.int32),
        scratch_shapes=[
            pltpu.VMEM((_BM * _NII, 1), jnp.float32),
            pltpu.VMEM((_BM * _NII, 1), jnp.float32),
            pltpu.VMEM((_BM * _NII, 1), jnp.float32),
        ],
        compiler_params=pltpu.CompilerParams(
            dimension_semantics=("parallel", "arbitrary", "arbitrary")),
    )(latent, emb, t, colf)


@functools.cache
def _sc_info():
    info = plsc.get_sparse_core_info()
    nw = info.num_cores * info.num_subcores
    return info.num_cores, nw, _N_E // nw


_HH = 128
_LANES = 16


def _sc_gather_st_body(table_hbm, idx_hbm, x_hbm, out_hbm, p_hbm,
                       idx_v, rows_v, xv, acc_v, sem):
    nc, _, bpw = _sc_info()
    wid = lax.axis_index("s") * nc + lax.axis_index("c")
    base = wid * bpw
    pltpu.sync_copy(idx_hbm.at[pl.ds(base, bpw)], idx_v)
    nvec = _HH * _E_DIM // _LANES
    acc = jnp.zeros((_LANES,), jnp.float32)

    for h in range(bpw // _HH):
        pltpu.async_copy(table_hbm.at[idx_v.at[pl.ds(h * _HH, _HH)]],
                         rows_v, sem).wait()
        off = (base + h * _HH) * _E_DIM
        pltpu.sync_copy(x_hbm.at[pl.ds(off, _HH * _E_DIM)], xv)

        def body(i, a):
            r = i // (_E_DIM // _LANES)
            c = (i % (_E_DIM // _LANES)) * _LANES
            xq = rows_v[r, pl.ds(c, _LANES)]
            xb = xv[pl.ds(i * _LANES, _LANES)]
            diff = xq - xb
            xv[pl.ds(i * _LANES, _LANES)] = xb + diff
            return a + diff * diff

        acc = lax.fori_loop(0, nvec, body, acc, unroll=4)
        pltpu.sync_copy(xv, out_hbm.at[pl.ds(off, _HH * _E_DIM)])

    acc_v[...] = acc
    pltpu.sync_copy(acc_v, p_hbm.at[wid])


def _sc_gather_st(emb, idx, latent_flat):
    _, nw, bpw = _sc_info()
    k = functools.partial(
        pl.kernel,
        out_type=[
            jax.ShapeDtypeStruct((_N_E * _E_DIM,), jnp.float32),
            jax.ShapeDtypeStruct((nw, _LANES), jnp.float32),
        ],
        mesh=plsc.VectorSubcoreMesh(core_axis_name="c", subcore_axis_name="s"),
        scratch_types=[
            pltpu.VMEM((bpw,), jnp.int32),
            pltpu.VMEM((_HH, _E_DIM), jnp.float32),
            pltpu.VMEM((_HH * _E_DIM,), jnp.float32),
            pltpu.VMEM((_LANES,), jnp.float32),
            pltpu.SemaphoreType.DMA,
        ],
    )(_sc_gather_st_body)
    return k(emb, idx, latent_flat)


def kernel(x, emb):
    latent = x.reshape(_N_E, _E_DIM)
    t = jnp.sum(emb * emb, axis=1)[None, :]
    colf = lax.iota(jnp.float32, _N_E)[None, :]
    idx2d = _argmin_indices(latent, emb, t, colf)
    idx = idx2d.reshape(_N_E)
    x_q_st_flat, partials = _sc_gather_st(emb, idx, x.reshape(-1))
    m = jnp.sum(partials) / jnp.float32(x.size)
    loss = m + jnp.float32(_BETA) * m
    return (x_q_st_flat.reshape(x.shape), loss, idx.reshape(x.shape[:-1]))


# speedup vs baseline: 1.2217x; 1.2217x over previous
import functools

import jax
import jax.numpy as jnp
from jax import lax
from jax.experimental import pallas as pl
from jax.experimental.pallas import tpu as pltpu
from jax.experimental.pallas import tpu_sc as plsc

_N_E = 8192
_E_DIM = 256
_BETA = 0.25

_BM = 2048
_BN = 2048
_NJ = _N_E // _BN


_NI = _N_E // _BM
_NIO = 1
_NII = _NI // _NIO


def _dist_argmin_kernel(l_ref, e_ref, t_ref, col_ref, out_ref,
                        s_ref, m_ref, i_ref):
    j = pl.program_id(1)
    row = pl.program_id(2) * _BM

    @pl.when(j == 0)
    def _():
        lb = l_ref[...]
        s_ref[pl.ds(row, _BM), :] = jnp.sum(lb * lb, axis=1, keepdims=True)

    lb = l_ref[...]
    mm2 = lax.dot_general(lb + lb, e_ref[...],
                          (((1,), (1,)), ((), ())),
                          preferred_element_type=jnp.float32)
    d = (s_ref[pl.ds(row, _BM), :] + t_ref[...]) - mm2
    bmin = jnp.min(d, axis=1, keepdims=True)
    bidxf = jnp.min(jnp.where(d == bmin, col_ref[...], jnp.float32(1e9)),
                    axis=1, keepdims=True)

    @pl.when(j == 0)
    def _():
        m_ref[pl.ds(row, _BM), :] = bmin
        i_ref[pl.ds(row, _BM), :] = bidxf

    @pl.when(j > 0)
    def _():
        upd = bmin < m_ref[pl.ds(row, _BM), :]
        m_ref[pl.ds(row, _BM), :] = jnp.where(upd, bmin, m_ref[pl.ds(row, _BM), :])
        i_ref[pl.ds(row, _BM), :] = jnp.where(upd, bidxf, i_ref[pl.ds(row, _BM), :])

    @pl.when(j == _NJ - 1)
    def _():
        out_ref[...] = i_ref[pl.ds(row, _BM), :].astype(jnp.int32)


def _argmin_indices(latent, emb, t, colf):
    return pl.pallas_call(
        _dist_argmin_kernel,
        grid=(_NIO, _NJ, _NII),
        in_specs=[
            pl.BlockSpec((_BM, _E_DIM), lambda io, j, ii: (io * _NII + ii, 0)),
            pl.BlockSpec((_BN, _E_DIM), lambda io, j, ii: (j, 0)),
            pl.BlockSpec((1, _BN), lambda io, j, ii: (0, j)),
            pl.BlockSpec((1, _BN), lambda io, j, ii: (0, j)),
        ],
        out_specs=pl.BlockSpec((_BM, 1), lambda io, j, ii: (io * _NII + ii, 0)),
        out_shape=jax.ShapeDtypeStruct((_N_E, 1), jnp.int32),
        scratch_shapes=[
            pltpu.VMEM((_BM * _NII, 1), jnp.float32),
            pltpu.VMEM((_BM * _NII, 1), jnp.float32),
            pltpu.VMEM((_BM * _NII, 1), jnp.float32),
        ],
        compiler_params=pltpu.CompilerParams(
            dimension_semantics=("parallel", "arbitrary", "arbitrary")),
    )(latent, emb, t, colf)


@functools.cache
def _sc_info():
    info = plsc.get_sparse_core_info()
    nw = info.num_cores * info.num_subcores
    return info.num_cores, nw, _N_E // nw


def _sc_gather_body(table_hbm, idx_hbm, out_hbm, idx_v, rows_v, sem):
    nc, _, _BPW = _sc_info()
    wid = lax.axis_index("s") * nc + lax.axis_index("c")
    base = wid * _BPW
    pltpu.sync_copy(idx_hbm.at[pl.ds(base, _BPW)], idx_v)
    cps = [
        pltpu.async_copy(table_hbm.at[idx_v.at[pl.ds(k * 128, 128)]],
                         rows_v.at[pl.ds(k * 128, 128)], sem)
        for k in range(_BPW // 128)
    ]
    for cp in cps:
        cp.wait()
    pltpu.sync_copy(rows_v, out_hbm.at[pl.ds(base, _BPW)])


def _sc_gather(emb, idx):
    _, _, bpw = _sc_info()
    k = functools.partial(
        pl.kernel,
        out_type=jax.ShapeDtypeStruct((_N_E, _E_DIM), jnp.float32),
        mesh=plsc.VectorSubcoreMesh(core_axis_name="c", subcore_axis_name="s"),
        scratch_types=[
            pltpu.VMEM((bpw,), jnp.int32),
            pltpu.VMEM((bpw, _E_DIM), jnp.float32),
            pltpu.SemaphoreType.DMA,
        ],
    )(_sc_gather_body)
    return k(emb, idx)


def _st_loss_kernel(x_ref, xq_ref, out_ref, p_ref):
    xb = x_ref[...]
    diff = xq_ref[...] - xb
    out_ref[...] = xb + diff
    p_ref[...] = jnp.sum(diff * diff, keepdims=True).reshape(1, 1, 1)


def _st_loss(latent, x_q):
    nb = 2
    bm = _N_E // nb
    return pl.pallas_call(
        _st_loss_kernel,
        grid=(nb,),
        in_specs=[
            pl.BlockSpec((bm, _E_DIM), lambda i: (i, 0)),
            pl.BlockSpec((bm, _E_DIM), lambda i: (i, 0)),
        ],
        out_specs=[
            pl.BlockSpec((bm, _E_DIM), lambda i: (i, 0)),
            pl.BlockSpec((1, 1, 1), lambda i: (i, 0, 0)),
        ],
        out_shape=[
            jax.ShapeDtypeStruct((_N_E, _E_DIM), jnp.float32),
            jax.ShapeDtypeStruct((nb, 1, 1), jnp.float32),
        ],
        compiler_params=pltpu.CompilerParams(
            dimension_semantics=("arbitrary",)),
    )(latent, x_q)


def kernel(x, emb):
    latent = x.reshape(_N_E, _E_DIM)
    t = jnp.sum(emb * emb, axis=1)[None, :]
    colf = lax.iota(jnp.float32, _N_E)[None, :]
    idx2d = _argmin_indices(latent, emb, t, colf)
    idx = idx2d.reshape(_N_E)
    x_q = _sc_gather(emb, idx)
    x_q_st, partials = _st_loss(latent, x_q)
    m = jnp.sum(partials) / jnp.float32(x.size)
    loss = m + jnp.float32(_BETA) * m
    return (x_q_st.reshape(x.shape), loss, idx.reshape(x.shape[:-1]))
